# baseline (speedup 1.0000x reference)
import functools

import jax
import jax.numpy as jnp
from jax import lax
from jax.experimental import pallas as pl
from jax.experimental.pallas import tpu as pltpu
from jax.experimental.pallas import tpu_sc as plsc

_NUM_USERS = 100000
_MAX_MEM = 16
_D = 64
_TOP_K = 4
_INV_TEMP = 10.0
_BATCH = 4096

_NC = 2
_NS = 16
_NW = _NC * _NS
_BPW = _BATCH // _NW
_CH = 32
_NCHUNK = _BPW // _CH


def _fast_rsqrt(x):
    i = plsc.bitcast(x, jnp.int32)
    i = jnp.int32(0x5F3759DF) - lax.shift_right_logical(i, 1)
    r = plsc.bitcast(i, jnp.float32)
    for _ in range(3):
        r = r * (1.5 - 0.5 * x * r * r)
    return r


def _retrieve_body(qn_hbm, keys_hbm, vals_hbm, uid_hbm, cnt_hbm, out_hbm,
                   uid_v, cnt_v, qn_v, keys_v, vidx_v, w_v, vsel_v, out_v, sem):
    wid = lax.axis_index("s") * _NC + lax.axis_index("c")
    base = wid * _BPW

    pltpu.sync_copy(uid_hbm.at[pl.ds(base, _BPW)], uid_v)
    pltpu.sync_copy(qn_hbm.at[pl.ds(base * _D, _BPW * _D)], qn_v)
    pltpu.async_copy(cnt_hbm.at[uid_v], cnt_v, sem).wait()

    iota = lax.iota(jnp.int32, 16)
    col_base = iota * _D
    first4 = iota < _TOP_K

    for ci in range(_NCHUNK):
        pltpu.async_copy(keys_hbm.at[uid_v.at[pl.ds(ci * _CH, _CH)]], keys_v,
                         sem).wait()

        def q_body(q, carry, ci=ci):
            qq = ci * _CH + q
            rows = jnp.broadcast_to(q, (16,)).astype(jnp.int32)
            qsplat = jnp.broadcast_to(qq, (16,)).astype(jnp.int32)
            dot = jnp.zeros((16,), jnp.float32)
            nrm = jnp.zeros((16,), jnp.float32)
            qoff = qq * _D
            for r in range(_D // 16):
                qblk = qn_v[pl.ds(qoff + 16 * r, 16)]
                for j in range(16):
                    kcol = plsc.load_gather(keys_v,
                                            [rows, col_base + (16 * r + j)])
                    dot = dot + kcol * qblk[j]
                    nrm = nrm + kcol * kcol
            cntv = plsc.load_gather(cnt_v, [qsplat])
            sims = dot * _fast_rsqrt(jnp.maximum(nrm, 1e-24))
            msims = jnp.where(iota < cntv, sims, jnp.float32(-1e9))
            vmax = jnp.max(msims)
            sk, sv = plsc.sort_key_val(msims, iota, descending=True)
            e = jnp.where(first4, jnp.exp((sk - vmax) * _INV_TEMP), 0.0)
            w = e / jnp.sum(e)
            vidx = plsc.load_gather(uid_v, [qsplat]) * _MAX_MEM + sv
            pos = iota + q * _TOP_K
            plsc.store_scatter(vidx_v, [pos], vidx, mask=first4)
            plsc.store_scatter(w_v, [pos], w, mask=first4)
            return carry

        lax.fori_loop(0, _CH, q_body, 0)

        pltpu.async_copy(vals_hbm.at[vidx_v], vsel_v, sem).wait()

        def b_body(q, carry):
            accs = [jnp.zeros((16,), jnp.float32) for _ in range(_D // 16)]
            for k in range(_TOP_K):
                row = q * _TOP_K + k
                rsp = jnp.broadcast_to(row, (16,)).astype(jnp.int32)
                wkv = plsc.load_gather(w_v, [rsp])
                for r in range(_D // 16):
                    v = plsc.load_gather(vsel_v, [rsp, iota + 16 * r])
                    accs[r] = accs[r] + wkv * v
            for r in range(_D // 16):
                out_v[pl.ds(q * _D + 16 * r, 16)] = accs[r]
            return carry

        lax.fori_loop(0, _CH, b_body, 0)

        pltpu.sync_copy(out_v,
                        out_hbm.at[pl.ds((base + ci * _CH) * _D, _CH * _D)])


_retrieve = functools.partial(
    pl.kernel,
    out_type=jax.ShapeDtypeStruct((_BATCH * _D,), jnp.float32),
    mesh=plsc.VectorSubcoreMesh(core_axis_name="c", subcore_axis_name="s"),
    compiler_params=pltpu.CompilerParams(use_tc_tiling_on_sc=False,
                                         needs_layout_passes=False),
    scratch_types=[
        pltpu.VMEM((_BPW,), jnp.int32),
        pltpu.VMEM((_BPW,), jnp.int32),
        pltpu.VMEM((_BPW * _D,), jnp.float32),
        pltpu.VMEM((_CH, _MAX_MEM * _D), jnp.float32),
        pltpu.VMEM((_CH * _TOP_K,), jnp.int32),
        pltpu.VMEM((_CH * _TOP_K,), jnp.float32),
        pltpu.VMEM((_CH * _TOP_K, _D), jnp.float32),
        pltpu.VMEM((_CH * _D,), jnp.float32),
        pltpu.SemaphoreType.DMA,
    ],
)(_retrieve_body)


def _qn_body(q_ref, wk_ref, o_ref):
    y = lax.dot_general(q_ref[...], wk_ref[...], (((1,), (1,)), ((), ())),
                        preferred_element_type=jnp.float32)
    n = jnp.sqrt(jnp.sum(y * y, axis=-1, keepdims=True))
    o_ref[...] = y / jnp.maximum(n, 1e-12)


_qn_call = pl.pallas_call(
    _qn_body,
    out_shape=jax.ShapeDtypeStruct((_BATCH, _D), jnp.float32),
)


def _proj_body(b_ref, wv_ref, o_ref):
    o_ref[...] = lax.dot_general(b_ref[...], wv_ref[...],
                                 (((1,), (1,)), ((), ())),
                                 preferred_element_type=jnp.float32)


_proj_call = pl.pallas_call(
    _proj_body,
    out_shape=jax.ShapeDtypeStruct((_BATCH, _D), jnp.float32),
)


def kernel(query, keys_buf, values_buf, W_key, W_val, episodic_scale,
           user_ids, memory_count):
    qn = _qn_call(query, W_key)
    keys2d = keys_buf.reshape(_NUM_USERS, _MAX_MEM * _D)
    vals2d = values_buf.reshape(_NUM_USERS * _MAX_MEM, _D)
    uid = user_ids.astype(jnp.int32)
    cnt = memory_count.astype(jnp.int32)
    blended = _retrieve(qn.reshape(-1), keys2d, vals2d, uid, cnt)
    blended = blended.reshape(_BATCH, _D)
    return _proj_call(blended, W_val * episodic_scale)
